# Initial kernel scaffold; baseline (speedup 1.0000x reference)
import jax
import jax.numpy as jnp
from jax.experimental import pallas as pl


def kernel(pos, edge_index, cW1, cb1, bg1, bb1, cW11, cb11, bg11, bb11, cW12, cb12, bg12, bb12, cW2, cb2, bg2, bb2, cW3, cb3, bg3, bb3, f0W, f0b, f1W, f1b, f2W, f2b, f3W, f3b):
    raise NotImplementedError("write your pallas kernel here")


# speedup vs baseline: 17.7371x; 17.7371x over previous
import functools

import jax
import jax.numpy as jnp
from jax import lax
from jax.experimental import pallas as pl
from jax.experimental.pallas import tpu as pltpu
from jax.experimental.pallas import tpu_sc as plsc

N = 50000
NPAD = 50176
TRASH = NPAD - N
E = 800000
EPAD = 819200
EB = EPAD // 128
WB = 112
NC, NS = 2, 16
R = 512
G = NPAD // R
F32 = jnp.float32

_mesh = plsc.VectorSubcoreMesh(core_axis_name="c", subcore_axis_name="s")
_sc_params = pltpu.CompilerParams(use_tc_tiling_on_sc=False)


def _zero_fill(buf, rows, d):
    def zz(i, _):
        for kk in range(d // 16):
            buf[i, pl.ds(kk * 16, 16)] = jnp.zeros((16,), F32)
        return 0
    lax.fori_loop(0, rows, zz, 0)


def _make_deg():
    BT = EB // (NC * NS)
    RPT = NPAD // NS

    @functools.partial(
        pl.kernel,
        out_type=jax.ShapeDtypeStruct((NC, NPAD, 16), F32),
        mesh=_mesh,
        compiler_params=_sc_params,
        scratch_types=[
            pltpu.VMEM_SHARED((NPAD, 16), F32),
            pltpu.VMEM((8, 128), jnp.int32),
            pltpu.VMEM((128, 16), F32),
            pltpu.VMEM((WB, 16), F32),
            pltpu.VMEM((WB, 16), F32),
        ],
    )
    def deg_kernel(colh, out, acc, colb, ones, zbuf, bbuf):
        c = lax.axis_index("c")
        s = lax.axis_index("s")
        w = c * NS + s
        _zero_fill(zbuf, WB, 16)

        def fill_ones(i, _):
            ones[i, pl.ds(0, 16)] = jnp.ones((16,), F32)
            return 0
        lax.fori_loop(0, 128, fill_ones, 0)

        def zacc(i, _):
            pltpu.sync_copy(zbuf, acc.at[pl.ds(s * RPT + i * WB, WB)])
            return 0
        lax.fori_loop(0, RPT // WB, zacc, 0)
        plsc.subcore_barrier()

        base = w * BT

        def strip(t, _):
            pltpu.sync_copy(colh.at[pl.ds(base + t * 8, 8)], colb)
            for j in range(8):
                pltpu.sync_copy(ones, acc.at[colb.at[j]], add=True)
            return 0
        lax.fori_loop(0, BT // 8, strip, 0)
        plsc.subcore_barrier()

        def wb(i, _):
            off = s * RPT + i * WB
            pltpu.sync_copy(acc.at[pl.ds(off, WB)], bbuf)
            pltpu.sync_copy(bbuf, out.at[c].at[pl.ds(off, WB)])
            return 0
        lax.fori_loop(0, RPT // WB, wb, 0)

    return deg_kernel


def _make_agg_plain(d, slab_by_core, SB):
    BT = EB // NS if slab_by_core else EB // (NS * NC)
    STRIPS = BT // SB
    RPT = NPAD // NS
    xs_shape = (NC, NPAD, d) if slab_by_core else (NPAD, d)

    @functools.partial(
        pl.kernel,
        out_type=jax.ShapeDtypeStruct((NC, NPAD, d), F32),
        mesh=_mesh,
        compiler_params=_sc_params,
        scratch_types=[
            pltpu.VMEM_SHARED((NPAD, d), F32),
            pltpu.VMEM((2, SB, 128), jnp.int32),
            pltpu.VMEM((2, SB, 128), jnp.int32),
            pltpu.VMEM((2, SB, 128, d), F32),
            pltpu.VMEM((WB, d), F32),
            pltpu.VMEM((WB, d), F32),
            pltpu.SemaphoreType.DMA,
        ],
    )
    def agg_kernel(xs, rowh, colh, out, acc, rowb, colb, gbuf, zbuf, bbuf, semg):
        c = lax.axis_index("c")
        s = lax.axis_index("s")
        _zero_fill(zbuf, WB, d)

        def zacc(i, _):
            pltpu.sync_copy(zbuf, acc.at[pl.ds(s * RPT + i * WB, WB)])
            return 0
        lax.fori_loop(0, RPT // WB, zacc, 0)
        plsc.subcore_barrier()

        base = (s if slab_by_core else c * NS + s) * BT
        xs_slab = xs.at[c] if slab_by_core else xs

        def load_idx(t, pb):
            pltpu.sync_copy(rowh.at[pl.ds(base + t * SB, SB)], rowb.at[pb])
            pltpu.sync_copy(colh.at[pl.ds(base + t * SB, SB)], colb.at[pb])

        def fire(pb):
            for j in range(SB):
                pltpu.async_copy(
                    xs_slab.at[rowb.at[pb].at[j]], gbuf.at[pb].at[j], semg)

        load_idx(0, 0)
        fire(0)

        def strip(t, _):
            pb = t % 2
            pn = (t + 1) % 2

            @pl.when(t < STRIPS - 1)
            def _():
                load_idx(t + 1, pn)

            for j in range(SB):
                pltpu.make_async_copy(
                    xs_slab.at[rowb.at[pb].at[j]], gbuf.at[pb].at[j], semg
                ).wait()

            @pl.when(t < STRIPS - 1)
            def _():
                fire(pn)

            for j in range(SB):
                pltpu.sync_copy(gbuf.at[pb].at[j], acc.at[colb.at[pb].at[j]],
                                add=True)
            return 0
        lax.fori_loop(0, STRIPS, strip, 0)
        plsc.subcore_barrier()

        def wb(i, _):
            off = s * RPT + i * WB
            pltpu.sync_copy(acc.at[pl.ds(off, WB)], bbuf)
            pltpu.sync_copy(bbuf, out.at[c].at[pl.ds(off, WB)])
            return 0
        lax.fori_loop(0, RPT // WB, wb, 0)

    return agg_kernel, xs_shape


def _prep_call(degp, pos_p):
    def body(dg_ref, pos_ref, dv_ref, u1_ref):
        i = pl.program_id(0)
        deg = dg_ref[0][:, 0:1] + dg_ref[1][:, 0:1] + 1.0
        dv = lax.rsqrt(deg)
        rows = lax.broadcasted_iota(jnp.int32, (R, 1), 0) + i * R
        dv = jnp.where(rows < N, dv, 0.0)
        dv_ref[...] = dv
        u1_ref[...] = dv * pos_ref[...]

    return pl.pallas_call(
        body,
        grid=(G,),
        in_specs=[
            pl.BlockSpec((NC, R, 16), lambda i: (0, i, 0)),
            pl.BlockSpec((R, 16), lambda i: (i, 0)),
        ],
        out_specs=[
            pl.BlockSpec((R, 1), lambda i: (i, 0)),
            pl.BlockSpec((R, 16), lambda i: (i, 0)),
        ],
        out_shape=[
            jax.ShapeDtypeStruct((NPAD, 1), F32),
            jax.ShapeDtypeStruct((NPAD, 16), F32),
        ],
    )(degp, pos_p)


def _t1_call(s, u, dinv, W, b, smode):
    d_in, d_out = W.shape
    nseg = 2 if smode == "cat3" else 1
    s = s if nseg == 2 else (s,)
    u = u if nseg == 2 else (u,)

    def body(*refs):
        (s_refs, u_refs, (dv_ref, w_ref, b_ref), (z_ref, st_ref)) = (
            refs[:nseg], refs[nseg:2 * nseg], refs[2 * nseg:2 * nseg + 3],
            refs[2 * nseg + 3:])
        i = pl.program_id(0)
        if smode == "sum2":
            sv = s_refs[0][0] + s_refs[0][1]
            uv = u_refs[0][...]
        elif smode == "cat2":
            sv = jnp.concatenate([s_refs[0][0], s_refs[0][1]], axis=1)
            uv = jnp.concatenate([u_refs[0][0], u_refs[0][1]], axis=1)
        else:
            sv = jnp.concatenate(
                [s_refs[0][0], s_refs[0][1], s_refs[1][0] + s_refs[1][1]],
                axis=1)
            uv = jnp.concatenate(
                [u_refs[0][0], u_refs[0][1], u_refs[1][...]], axis=1)
        a = dv_ref[...] * (sv + uv)
        z = jnp.dot(a, w_ref[...], preferred_element_type=F32) + b_ref[...]
        rows = lax.broadcasted_iota(jnp.int32, (R, 1), 0) + i * R
        zm = jnp.where(rows < N, z, 0.0)
        z_ref[...] = z

        @pl.when(i == 0)
        def _():
            st_ref[...] = jnp.zeros_like(st_ref)
        st_ref[...] += jnp.concatenate(
            [jnp.sum(zm, 0, keepdims=True),
             jnp.sum(zm * zm, 0, keepdims=True)], 0)

    half = d_in // 2
    stk = pl.BlockSpec((NC, R, 32), lambda i: (0, i, 0))
    if smode == "sum2":
        s_specs = [pl.BlockSpec((NC, R, d_in), lambda i: (0, i, 0))]
        u_specs = [pl.BlockSpec((R, d_in), lambda i: (i, 0))]
    elif smode == "cat2":
        s_specs = [pl.BlockSpec((NC, R, half), lambda i: (0, i, 0))]
        u_specs = [pl.BlockSpec((NC, R, half), lambda i: (0, i, 0))]
    else:
        s_specs = [stk, stk]
        u_specs = [stk, pl.BlockSpec((R, 32), lambda i: (i, 0))]

    return pl.pallas_call(
        body,
        grid=(G,),
        in_specs=s_specs + u_specs + [
            pl.BlockSpec((R, 1), lambda i: (i, 0)),
            pl.BlockSpec((d_in, d_out), lambda i: (0, 0)),
            pl.BlockSpec((1, d_out), lambda i: (0, 0)),
        ],
        out_specs=[
            pl.BlockSpec((R, d_out), lambda i: (i, 0)),
            pl.BlockSpec((2, d_out), lambda i: (0, 0)),
        ],
        out_shape=[
            jax.ShapeDtypeStruct((NPAD, d_out), F32),
            jax.ShapeDtypeStruct((2, d_out), F32),
        ],
    )(*s, *u, dinv, W, b)


def _t2_call(z, st, g, bb, dinv, slabs):
    d_out = z.shape[1]
    half = d_out // 2

    def body(z_ref, st_ref, g_ref, b_ref, dv_ref, *refs):
        i = pl.program_id(0)
        stv = st_ref[...]
        m = stv[0:1] / N
        v = stv[1:2] / N - m * m
        inv = lax.rsqrt(v + 1e-5)
        x = jnp.maximum((z_ref[...] - m) * inv * g_ref[...] + b_ref[...], 0.0)
        un = dv_ref[...] * x
        rows = lax.broadcasted_iota(jnp.int32, (R, 1), 0) + i * R
        un = jnp.where(rows < N, un, 0.0)
        if slabs == 1:
            refs[0][...] = un
        elif slabs == 2:
            refs[0][0] = un[:, :half]
            refs[0][1] = un[:, half:]
        else:
            refs[0][0] = un[:, :32]
            refs[0][1] = un[:, 32:64]
            refs[1][...] = un[:, 64:96]

    if slabs == 1:
        o_specs = pl.BlockSpec((R, d_out), lambda i: (i, 0))
        o_shapes = jax.ShapeDtypeStruct((NPAD, d_out), F32)
    elif slabs == 2:
        o_specs = pl.BlockSpec((NC, R, half), lambda i: (0, i, 0))
        o_shapes = jax.ShapeDtypeStruct((NC, NPAD, half), F32)
    else:
        o_specs = [pl.BlockSpec((NC, R, 32), lambda i: (0, i, 0)),
                   pl.BlockSpec((R, 32), lambda i: (i, 0))]
        o_shapes = [jax.ShapeDtypeStruct((NC, NPAD, 32), F32),
                    jax.ShapeDtypeStruct((NPAD, 32), F32)]

    return pl.pallas_call(
        body,
        grid=(G,),
        in_specs=[
            pl.BlockSpec((R, d_out), lambda i: (i, 0)),
            pl.BlockSpec((2, d_out), lambda i: (0, 0)),
            pl.BlockSpec((1, d_out), lambda i: (0, 0)),
            pl.BlockSpec((1, d_out), lambda i: (0, 0)),
            pl.BlockSpec((R, 1), lambda i: (i, 0)),
        ],
        out_specs=o_specs,
        out_shape=o_shapes,
    )(z, st, g, bb, dinv)


def _t2max_call(z, st, g, bb, f0W, f0b):
    d_out = z.shape[1]

    def body(z_ref, st_ref, g_ref, b_ref, w_ref, fb_ref, mx_ref):
        i = pl.program_id(0)
        stv = st_ref[...]
        m = stv[0:1] / N
        v = stv[1:2] / N - m * m
        inv = lax.rsqrt(v + 1e-5)
        x = jnp.maximum((z_ref[...] - m) * inv * g_ref[...] + b_ref[...], 0.0)
        y = jnp.maximum(
            jnp.dot(x, w_ref[...], preferred_element_type=F32) + fb_ref[...],
            0.0)
        rows = lax.broadcasted_iota(jnp.int32, (R, 1), 0) + i * R
        y = jnp.where(rows < N, y, -3.0e38)
        bm = jnp.max(y, axis=0, keepdims=True)

        @pl.when(i == 0)
        def _():
            mx_ref[...] = jnp.full_like(mx_ref, -3.0e38)
        mx_ref[...] = jnp.maximum(mx_ref[...], bm)

    return pl.pallas_call(
        body,
        grid=(G,),
        in_specs=[
            pl.BlockSpec((R, d_out), lambda i: (i, 0)),
            pl.BlockSpec((2, d_out), lambda i: (0, 0)),
            pl.BlockSpec((1, d_out), lambda i: (0, 0)),
            pl.BlockSpec((1, d_out), lambda i: (0, 0)),
            pl.BlockSpec((d_out, 128), lambda i: (0, 0)),
            pl.BlockSpec((1, 128), lambda i: (0, 0)),
        ],
        out_specs=pl.BlockSpec((1, 128), lambda i: (0, 0)),
        out_shape=jax.ShapeDtypeStruct((1, 128), F32),
    )(z, st, g, bb, f0W, f0b)


def _head_call(mx, f1W, f1b, f2W, f2b, f3W, f3b):
    def body(x_ref, w1, b1, w2, b2, w3, b3, o_ref):
        y = jnp.maximum(
            jnp.dot(x_ref[...], w1[...], preferred_element_type=F32) + b1[...],
            0.0)
        y = jnp.maximum(
            jnp.dot(y, w2[...], preferred_element_type=F32) + b2[...], 0.0)
        y = jnp.dot(y, w3[...], preferred_element_type=F32) + b3[...]
        nrm = jnp.maximum(jnp.sqrt(jnp.sum(y * y)), 1e-12)
        o_ref[...] = y / nrm

    return pl.pallas_call(
        body,
        out_shape=jax.ShapeDtypeStruct((1, 128), F32),
    )(mx, f1W, f1b, f2W, f2b, f3W, f3b)


_deg_k = _make_deg()
_aggA16, _ = _make_agg_plain(16, False, 8)
_aggA32, _ = _make_agg_plain(32, False, 2)
_aggC32, _ = _make_agg_plain(32, True, 2)


def kernel(pos, edge_index, cW1, cb1, bg1, bb1, cW11, cb11, bg11, bb11,
           cW12, cb12, bg12, bb12, cW2, cb2, bg2, bb2, cW3, cb3, bg3, bb3,
           f0W, f0b, f1W, f1b, f2W, f2b, f3W, f3b):
    row = edge_index[0]
    col = edge_index[1]
    k = jnp.arange(EPAD - E, dtype=jnp.int32)
    row2d = jnp.concatenate([row, N + (k % TRASH)]).reshape(EB, 128)
    col2d = jnp.concatenate([col, N + ((k * 7) % TRASH)]).reshape(EB, 128)
    pos_p = jnp.zeros((NPAD, 16), F32).at[:N, :3].set(pos)

    W1p = jnp.zeros((16, 16), F32).at[:3].set(cW1)
    W2p = jnp.zeros((64, 96), F32).at[:, :94].set(cW2)
    W3p = jnp.zeros((96, 256), F32).at[:94].set(cW3)
    b2p = jnp.zeros((1, 96), F32).at[:, :94].set(cb2.reshape(1, -1))
    g2p = jnp.zeros((1, 96), F32).at[:, :94].set(bg2.reshape(1, -1))
    bb2p = jnp.zeros((1, 96), F32).at[:, :94].set(bb2.reshape(1, -1))

    degp = _deg_k(col2d)
    dinv, u1 = _prep_call(degp, pos_p)

    s1 = _aggA16(u1, row2d, col2d)
    z1, st1 = _t1_call(s1, u1, dinv, W1p, cb1.reshape(1, -1), "sum2")
    u2 = _t2_call(z1, st1, bg1.reshape(1, -1), bb1.reshape(1, -1), dinv, 1)

    s2 = _aggA16(u2, row2d, col2d)
    z2, st2 = _t1_call(s2, u2, dinv, cW11, cb11.reshape(1, -1), "sum2")
    u3 = _t2_call(z2, st2, bg11.reshape(1, -1), bb11.reshape(1, -1), dinv, 1)

    s3 = _aggA32(u3, row2d, col2d)
    z3, st3 = _t1_call(s3, u3, dinv, cW12, cb12.reshape(1, -1), "sum2")
    u4 = _t2_call(z3, st3, bg12.reshape(1, -1), bb12.reshape(1, -1), dinv, 2)

    s4 = _aggC32(u4, row2d, col2d)
    z4, st4 = _t1_call(s4, u4, dinv, W2p, b2p, "cat2")
    u5c, u5a = _t2_call(z4, st4, g2p, bb2p, dinv, 3)

    s5c = _aggC32(u5c, row2d, col2d)
    s5a = _aggA32(u5a, row2d, col2d)
    z5, st5 = _t1_call((s5c, s5a), (u5c, u5a), dinv, W3p,
                       cb3.reshape(1, -1), "cat3")
    mx = _t2max_call(z5, st5, bg3.reshape(1, -1), bb3.reshape(1, -1),
                     f0W, f0b.reshape(1, -1))

    return _head_call(mx, f1W, f1b.reshape(1, -1), f2W, f2b.reshape(1, -1),
                      f3W, f3b.reshape(1, -1))


# speedup vs baseline: 18.7173x; 1.0553x over previous
import functools

import jax
import jax.numpy as jnp
from jax import lax
from jax.experimental import pallas as pl
from jax.experimental.pallas import tpu as pltpu
from jax.experimental.pallas import tpu_sc as plsc

N = 50000
NPAD = 50176
TRASH = NPAD - N
E = 800000
EPAD = 819200
EB = EPAD // 128
WB = 112
NC, NS = 2, 16
R = 512
G = NPAD // R
F32 = jnp.float32

_mesh = plsc.VectorSubcoreMesh(core_axis_name="c", subcore_axis_name="s")
_sc_params = pltpu.CompilerParams(use_tc_tiling_on_sc=False)


def _zero_fill(buf, rows, d):
    def zz(i, _):
        for kk in range(d // 16):
            buf[i, pl.ds(kk * 16, 16)] = jnp.zeros((16,), F32)
        return 0
    lax.fori_loop(0, rows, zz, 0)


def _make_deg():
    BT = EB // (NC * NS)
    RPT = NPAD // NS

    @functools.partial(
        pl.kernel,
        out_type=jax.ShapeDtypeStruct((NC, NPAD, 16), F32),
        mesh=_mesh,
        compiler_params=_sc_params,
        scratch_types=[
            pltpu.VMEM_SHARED((NPAD, 16), F32),
            pltpu.VMEM((8, 128), jnp.int32),
            pltpu.VMEM((128, 16), F32),
            pltpu.VMEM((WB, 16), F32),
            pltpu.VMEM((WB, 16), F32),
        ],
    )
    def deg_kernel(colh, out, acc, colb, ones, zbuf, bbuf):
        c = lax.axis_index("c")
        s = lax.axis_index("s")
        w = c * NS + s
        _zero_fill(zbuf, WB, 16)

        def fill_ones(i, _):
            ones[i, pl.ds(0, 16)] = jnp.ones((16,), F32)
            return 0
        lax.fori_loop(0, 128, fill_ones, 0)

        def zacc(i, _):
            pltpu.sync_copy(zbuf, acc.at[pl.ds(s * RPT + i * WB, WB)])
            return 0
        lax.fori_loop(0, RPT // WB, zacc, 0)
        plsc.subcore_barrier()

        base = w * BT

        def strip(t, _):
            pltpu.sync_copy(colh.at[pl.ds(base + t * 8, 8)], colb)
            for j in range(8):
                pltpu.sync_copy(ones, acc.at[colb.at[j]], add=True)
            return 0
        lax.fori_loop(0, BT // 8, strip, 0)
        plsc.subcore_barrier()

        def wb(i, _):
            off = s * RPT + i * WB
            pltpu.sync_copy(acc.at[pl.ds(off, WB)], bbuf)
            pltpu.sync_copy(bbuf, out.at[c].at[pl.ds(off, WB)])
            return 0
        lax.fori_loop(0, RPT // WB, wb, 0)

    return deg_kernel


def _make_agg16(edge_split):
    SB = 8
    BT = EB // (NS * NC) if edge_split else EB // NS
    STRIPS = BT // SB
    RPT = NPAD // NS
    xs_shape = (NPAD, 16) if edge_split else (NC, NPAD, 16)

    @functools.partial(
        pl.kernel,
        out_type=jax.ShapeDtypeStruct((NC, NPAD, 16), F32),
        mesh=_mesh,
        compiler_params=_sc_params,
        scratch_types=[
            pltpu.VMEM_SHARED((NPAD, 16), F32),
            pltpu.VMEM((2, SB, 128), jnp.int32),
            pltpu.VMEM((2, SB, 128), jnp.int32),
            pltpu.VMEM((2, SB, 128, 16), F32),
            pltpu.VMEM((WB, 16), F32),
            pltpu.VMEM((WB, 16), F32),
            pltpu.SemaphoreType.DMA,
            pltpu.SemaphoreType.DMA,
        ],
    )
    def agg_kernel(xs, rowh, colh, out, acc, rowb, colb, gbuf, zbuf, bbuf,
                   semg, sems):
        c = lax.axis_index("c")
        s = lax.axis_index("s")
        _zero_fill(zbuf, WB, 16)

        def zacc(i, _):
            pltpu.sync_copy(zbuf, acc.at[pl.ds(s * RPT + i * WB, WB)])
            return 0
        lax.fori_loop(0, RPT // WB, zacc, 0)
        plsc.subcore_barrier()

        base = (c * NS + s if edge_split else s) * BT
        xs_slab = xs if edge_split else xs.at[c]

        def load_idx(t, pb):
            pltpu.sync_copy(rowh.at[pl.ds(base + t * SB, SB)], rowb.at[pb])
            pltpu.sync_copy(colh.at[pl.ds(base + t * SB, SB)], colb.at[pb])

        def fire_gathers(pb):
            for j in range(SB):
                pltpu.async_copy(
                    xs_slab.at[rowb.at[pb].at[j]], gbuf.at[pb].at[j], semg)

        def fire_scatters(pb):
            for j in range(SB):
                pltpu.async_copy(
                    gbuf.at[pb].at[j], acc.at[colb.at[pb].at[j]], sems,
                    add=True)

        def wait_gathers(pb):
            for j in range(SB):
                pltpu.make_async_copy(
                    xs_slab.at[rowb.at[pb].at[j]], gbuf.at[pb].at[j], semg
                ).wait()

        def wait_scatters(pb):
            for j in range(SB):
                pltpu.make_async_copy(
                    gbuf.at[pb].at[j], acc.at[colb.at[pb].at[j]], sems
                ).wait()

        load_idx(0, 0)
        fire_gathers(0)

        def strip(t, _):
            pb = t % 2
            pn = (t + 1) % 2

            @pl.when(t < STRIPS - 1)
            def _():
                load_idx(t + 1, pn)
            wait_gathers(pb)

            @pl.when(t > 0)
            def _():
                wait_scatters(pn)

            @pl.when(t < STRIPS - 1)
            def _():
                fire_gathers(pn)
            fire_scatters(pb)
            return 0
        lax.fori_loop(0, STRIPS, strip, 0)
        wait_scatters((STRIPS - 1) % 2)
        plsc.subcore_barrier()

        def wb(i, _):
            off = s * RPT + i * WB
            pltpu.sync_copy(acc.at[pl.ds(off, WB)], bbuf)
            pltpu.sync_copy(bbuf, out.at[c].at[pl.ds(off, WB)])
            return 0
        lax.fori_loop(0, RPT // WB, wb, 0)

    return agg_kernel, xs_shape


def _prep_call(degp, pos_p):
    def body(dg_ref, pos_ref, dv_ref, u1_ref):
        i = pl.program_id(0)
        deg = dg_ref[0][:, 0:1] + dg_ref[1][:, 0:1] + 1.0
        dv = lax.rsqrt(deg)
        rows = lax.broadcasted_iota(jnp.int32, (R, 1), 0) + i * R
        dv = jnp.where(rows < N, dv, 0.0)
        dv_ref[...] = dv
        u1_ref[...] = dv * pos_ref[...]

    return pl.pallas_call(
        body,
        grid=(G,),
        in_specs=[
            pl.BlockSpec((NC, R, 16), lambda i: (0, i, 0)),
            pl.BlockSpec((R, 16), lambda i: (i, 0)),
        ],
        out_specs=[
            pl.BlockSpec((R, 1), lambda i: (i, 0)),
            pl.BlockSpec((R, 16), lambda i: (i, 0)),
        ],
        out_shape=[
            jax.ShapeDtypeStruct((NPAD, 1), F32),
            jax.ShapeDtypeStruct((NPAD, 16), F32),
        ],
    )(degp, pos_p)


def _t1_call(s, u, dinv, W, b, smode):
    d_in, d_out = W.shape
    if smode == "sum2":
        s, u = (s,), (u,)
    k = len(s)

    def body(*refs):
        s_refs = refs[:k]
        u_refs = refs[k:2 * k]
        dv_ref, w_ref, b_ref, z_ref, st_ref = refs[2 * k:]
        i = pl.program_id(0)
        if smode == "sum2":
            sv = s_refs[0][0] + s_refs[0][1]
            uv = u_refs[0][...]
        else:
            sv = jnp.concatenate(
                [sr[cc] for sr in s_refs for cc in range(NC)], axis=1)
            uv = jnp.concatenate(
                [ur[cc] for ur in u_refs for cc in range(NC)], axis=1)
        a = dv_ref[...] * (sv + uv)
        z = jnp.dot(a, w_ref[...], preferred_element_type=F32) + b_ref[...]
        rows = lax.broadcasted_iota(jnp.int32, (R, 1), 0) + i * R
        zm = jnp.where(rows < N, z, 0.0)
        z_ref[...] = z

        @pl.when(i == 0)
        def _():
            st_ref[...] = jnp.zeros_like(st_ref)
        st_ref[...] += jnp.concatenate(
            [jnp.sum(zm, 0, keepdims=True),
             jnp.sum(zm * zm, 0, keepdims=True)], 0)

    stk = pl.BlockSpec((NC, R, 16), lambda i: (0, i, 0))
    if smode == "sum2":
        s_specs = [pl.BlockSpec((NC, R, d_in), lambda i: (0, i, 0))]
        u_specs = [pl.BlockSpec((R, d_in), lambda i: (i, 0))]
    else:
        s_specs = [stk] * k
        u_specs = [stk] * k

    return pl.pallas_call(
        body,
        grid=(G,),
        in_specs=s_specs + u_specs + [
            pl.BlockSpec((R, 1), lambda i: (i, 0)),
            pl.BlockSpec((d_in, d_out), lambda i: (0, 0)),
            pl.BlockSpec((1, d_out), lambda i: (0, 0)),
        ],
        out_specs=[
            pl.BlockSpec((R, d_out), lambda i: (i, 0)),
            pl.BlockSpec((2, d_out), lambda i: (0, 0)),
        ],
        out_shape=[
            jax.ShapeDtypeStruct((NPAD, d_out), F32),
            jax.ShapeDtypeStruct((2, d_out), F32),
        ],
    )(*s, *u, dinv, W, b)


def _t2_call(z, st, g, bb, dinv, k):
    d_out = z.shape[1]

    def body(z_ref, st_ref, g_ref, b_ref, dv_ref, *refs):
        i = pl.program_id(0)
        stv = st_ref[...]
        m = stv[0:1] / N
        v = stv[1:2] / N - m * m
        inv = lax.rsqrt(v + 1e-5)
        x = jnp.maximum((z_ref[...] - m) * inv * g_ref[...] + b_ref[...], 0.0)
        un = dv_ref[...] * x
        rows = lax.broadcasted_iota(jnp.int32, (R, 1), 0) + i * R
        un = jnp.where(rows < N, un, 0.0)
        if k == 0:
            refs[0][...] = un
        else:
            for q in range(k):
                refs[q][0] = un[:, 32 * q:32 * q + 16]
                refs[q][1] = un[:, 32 * q + 16:32 * q + 32]

    if k == 0:
        o_specs = pl.BlockSpec((R, d_out), lambda i: (i, 0))
        o_shapes = jax.ShapeDtypeStruct((NPAD, d_out), F32)
    else:
        o_specs = [pl.BlockSpec((NC, R, 16), lambda i: (0, i, 0))] * k
        o_shapes = [jax.ShapeDtypeStruct((NC, NPAD, 16), F32)] * k

    return pl.pallas_call(
        body,
        grid=(G,),
        in_specs=[
            pl.BlockSpec((R, d_out), lambda i: (i, 0)),
            pl.BlockSpec((2, d_out), lambda i: (0, 0)),
            pl.BlockSpec((1, d_out), lambda i: (0, 0)),
            pl.BlockSpec((1, d_out), lambda i: (0, 0)),
            pl.BlockSpec((R, 1), lambda i: (i, 0)),
        ],
        out_specs=o_specs,
        out_shape=o_shapes,
    )(z, st, g, bb, dinv)


def _t2max_call(z, st, g, bb, f0W, f0b):
    d_out = z.shape[1]

    def body(z_ref, st_ref, g_ref, b_ref, w_ref, fb_ref, mx_ref):
        i = pl.program_id(0)
        stv = st_ref[...]
        m = stv[0:1] / N
        v = stv[1:2] / N - m * m
        inv = lax.rsqrt(v + 1e-5)
        x = jnp.maximum((z_ref[...] - m) * inv * g_ref[...] + b_ref[...], 0.0)
        y = jnp.maximum(
            jnp.dot(x, w_ref[...], preferred_element_type=F32) + fb_ref[...],
            0.0)
        rows = lax.broadcasted_iota(jnp.int32, (R, 1), 0) + i * R
        y = jnp.where(rows < N, y, -3.0e38)
        bm = jnp.max(y, axis=0, keepdims=True)

        @pl.when(i == 0)
        def _():
            mx_ref[...] = jnp.full_like(mx_ref, -3.0e38)
        mx_ref[...] = jnp.maximum(mx_ref[...], bm)

    return pl.pallas_call(
        body,
        grid=(G,),
        in_specs=[
            pl.BlockSpec((R, d_out), lambda i: (i, 0)),
            pl.BlockSpec((2, d_out), lambda i: (0, 0)),
            pl.BlockSpec((1, d_out), lambda i: (0, 0)),
            pl.BlockSpec((1, d_out), lambda i: (0, 0)),
            pl.BlockSpec((d_out, 128), lambda i: (0, 0)),
            pl.BlockSpec((1, 128), lambda i: (0, 0)),
        ],
        out_specs=pl.BlockSpec((1, 128), lambda i: (0, 0)),
        out_shape=jax.ShapeDtypeStruct((1, 128), F32),
    )(z, st, g, bb, f0W, f0b)


def _head_call(mx, f1W, f1b, f2W, f2b, f3W, f3b):
    def body(x_ref, w1, b1, w2, b2, w3, b3, o_ref):
        y = jnp.maximum(
            jnp.dot(x_ref[...], w1[...], preferred_element_type=F32) + b1[...],
            0.0)
        y = jnp.maximum(
            jnp.dot(y, w2[...], preferred_element_type=F32) + b2[...], 0.0)
        y = jnp.dot(y, w3[...], preferred_element_type=F32) + b3[...]
        nrm = jnp.maximum(jnp.sqrt(jnp.sum(y * y)), 1e-12)
        o_ref[...] = y / nrm

    return pl.pallas_call(
        body,
        out_shape=jax.ShapeDtypeStruct((1, 128), F32),
    )(mx, f1W, f1b, f2W, f2b, f3W, f3b)


_deg_k = _make_deg()
_aggE, _ = _make_agg16(True)
_aggS, _ = _make_agg16(False)


def kernel(pos, edge_index, cW1, cb1, bg1, bb1, cW11, cb11, bg11, bb11,
           cW12, cb12, bg12, bb12, cW2, cb2, bg2, bb2, cW3, cb3, bg3, bb3,
           f0W, f0b, f1W, f1b, f2W, f2b, f3W, f3b):
    row = edge_index[0]
    col = edge_index[1]
    k = jnp.arange(EPAD - E, dtype=jnp.int32)
    row2d = jnp.concatenate([row, N + (k % TRASH)]).reshape(EB, 128)
    col2d = jnp.concatenate([col, N + ((k * 7) % TRASH)]).reshape(EB, 128)
    pos_p = jnp.zeros((NPAD, 16), F32).at[:N, :3].set(pos)

    W1p = jnp.zeros((16, 16), F32).at[:3].set(cW1)
    W2p = jnp.zeros((64, 96), F32).at[:, :94].set(cW2)
    W3p = jnp.zeros((96, 256), F32).at[:94].set(cW3)
    b2p = jnp.zeros((1, 96), F32).at[:, :94].set(cb2.reshape(1, -1))
    g2p = jnp.zeros((1, 96), F32).at[:, :94].set(bg2.reshape(1, -1))
    bb2p = jnp.zeros((1, 96), F32).at[:, :94].set(bb2.reshape(1, -1))

    degp = _deg_k(col2d)
    dinv, u1 = _prep_call(degp, pos_p)

    s1 = _aggE(u1, row2d, col2d)
    z1, st1 = _t1_call(s1, u1, dinv, W1p, cb1.reshape(1, -1), "sum2")
    u2 = _t2_call(z1, st1, bg1.reshape(1, -1), bb1.reshape(1, -1), dinv, 0)

    s2 = _aggE(u2, row2d, col2d)
    z2, st2 = _t1_call(s2, u2, dinv, cW11, cb11.reshape(1, -1), "sum2")
    (u3,) = _t2_call(z2, st2, bg11.reshape(1, -1), bb11.reshape(1, -1),
                     dinv, 1)

    s3 = _aggS(u3, row2d, col2d)
    z3, st3 = _t1_call((s3,), (u3,), dinv, cW12, cb12.reshape(1, -1), "catk")
    u4a, u4b = _t2_call(z3, st3, bg12.reshape(1, -1), bb12.reshape(1, -1),
                        dinv, 2)

    s4a = _aggS(u4a, row2d, col2d)
    s4b = _aggS(u4b, row2d, col2d)
    z4, st4 = _t1_call((s4a, s4b), (u4a, u4b), dinv, W2p, b2p, "catk")
    u5a, u5b, u5c = _t2_call(z4, st4, g2p, bb2p, dinv, 3)

    s5a = _aggS(u5a, row2d, col2d)
    s5b = _aggS(u5b, row2d, col2d)
    s5c = _aggS(u5c, row2d, col2d)
    z5, st5 = _t1_call((s5a, s5b, s5c), (u5a, u5b, u5c), dinv, W3p,
                       cb3.reshape(1, -1), "catk")
    mx = _t2max_call(z5, st5, bg3.reshape(1, -1), bb3.reshape(1, -1),
                     f0W, f0b.reshape(1, -1))

    return _head_call(mx, f1W, f1b.reshape(1, -1), f2W, f2b.reshape(1, -1),
                      f3W, f3b.reshape(1, -1))
